# baseline (speedup 1.0000x reference)
import functools

import jax
import jax.numpy as jnp
from jax import lax
from jax.experimental import pallas as pl
from jax.experimental.pallas import tpu as pltpu
from jax.experimental.pallas import tpu_sc as plsc

_K = 8


def _tc1_body(x_ref, wq_ref, bq_ref, wk_ref, bk_ref, wv_ref, bv_ref,
              v_ref, idx_ref, w_ref, *, n, d, k):
    xb = x_ref[0]
    inv_s = 1.0 / (d ** 0.5)
    qT = (jnp.dot(wq_ref[...], xb, preferred_element_type=jnp.float32)
          + bq_ref[...]) * inv_s
    kT = jnp.dot(wk_ref[...], xb, preferred_element_type=jnp.float32) + bk_ref[...]
    v_rows = lax.dot_general(xb, wv_ref[...], (((0,), (1,)), ((), ())),
                             preferred_element_type=jnp.float32) + bv_ref[...]
    v_ref[0] = jnp.concatenate(
        [v_rows, jnp.zeros((n, 128 - v_rows.shape[1]), jnp.float32)], axis=1)

    sim = lax.dot_general(qT, kT, (((0,), (0,)), ((), ())),
                          preferred_element_type=jnp.float32)
    row = lax.broadcasted_iota(jnp.int32, (n, n), 0)
    col = lax.broadcasted_iota(jnp.int32, (n, n), 1)
    sim = jnp.where(row == col, sim - 1e9, sim)

    colf = lax.broadcasted_iota(jnp.int32, (n, 1), 0).astype(jnp.float32)
    work = sim
    m0 = None
    idxs = []
    ws = []
    for it in range(k):
        m = jnp.max(work, axis=1, keepdims=True)
        if it == 0:
            m0 = m
        hit = work == m
        onehot = jnp.where(hit, 1.0, 0.0)
        idxs.append(lax.dot_general(onehot, colf, (((1,), (0,)), ((), ())),
                                    precision=lax.Precision.HIGHEST,
                                    preferred_element_type=jnp.float32))
        ws.append(jnp.exp(m - m0))
        work = jnp.where(hit, -jnp.inf, work)

    idx_local = jnp.concatenate(idxs, axis=1)
    w = jnp.concatenate(ws, axis=1)
    w = w * (1.0 / jnp.sum(w, axis=1, keepdims=True))
    base = pl.program_id(0) * n
    idx_i = jnp.clip(idx_local.astype(jnp.int32), 0, n - 1)
    idx_ref[0] = idx_i + base
    w_ref[0] = w


def _make_sc_combine(rows_total, c, k, nw):
    rb = rows_total // nw
    chunk = 64
    nchunks = rb // chunk
    mesh = plsc.VectorSubcoreMesh(core_axis_name="c", subcore_axis_name="s")

    @functools.partial(
        pl.kernel, mesh=mesh,
        out_type=jax.ShapeDtypeStruct((rows_total, 128), jnp.float32),
        scratch_types=[
            pltpu.VMEM((rb * k,), jnp.int32),
            pltpu.VMEM((rb * k,), jnp.float32),
            pltpu.VMEM((chunk * k, 128), jnp.float32),
            pltpu.VMEM((chunk, 128), jnp.float32),
            pltpu.SemaphoreType.DMA,
        ],
    )
    def sc_combine(idx_hbm, w_hbm, v_hbm, out_hbm, idx_v, w_v, rows_v, acc_v, sem):
        wid = lax.axis_index("s") * 2 + lax.axis_index("c")
        base = wid * rb
        pltpu.sync_copy(idx_hbm.at[pl.ds(base * k, rb * k)], idx_v)
        pltpu.sync_copy(w_hbm.at[pl.ds(base * k, rb * k)], w_v)
        for ch in range(nchunks):
            pltpu.async_copy(
                v_hbm.at[idx_v.at[pl.ds(ch * chunk * k, chunk * k)]],
                rows_v, sem).wait()

            def row_body(r2, carry):
                wpair = w_v[pl.ds((ch * chunk // 2 + r2) * 16, 16)]
                for sub in range(2):
                    r = r2 * 2 + sub
                    for g in range(128 // 16):
                        acc = jnp.zeros((16,), jnp.float32)
                        for kk in range(k):
                            lane = jnp.full((16,), sub * k + kk, jnp.int32)
                            wvec = lax.gather(
                                wpair, lane[:, None],
                                dimension_numbers=lax.GatherDimensionNumbers(
                                    offset_dims=(), collapsed_slice_dims=(0,),
                                    start_index_map=(0,)),
                                slice_sizes=(1,),
                                mode=lax.GatherScatterMode.PROMISE_IN_BOUNDS)
                            acc = acc + wvec * rows_v[r * k + kk,
                                                      pl.ds(g * 16, 16)]
                        acc_v[r, pl.ds(g * 16, 16)] = acc
                return carry

            lax.fori_loop(0, chunk // 2, row_body, 0)
            pltpu.sync_copy(acc_v,
                            out_hbm.at[pl.ds(base + ch * chunk, chunk)])

    return sc_combine


def _tc3_body(x_ref, scale_ref, comb_ref, o_ref, *, n, c):
    xb = x_ref[0]
    r = lax.broadcasted_iota(jnp.int32, (c, c), 0)
    cc = lax.broadcasted_iota(jnp.int32, (c, c), 1)
    eye = jnp.where(r == cc, 1.0, 0.0)
    comb_t = lax.dot_general(eye, comb_ref[0][:, :c], (((1,), (1,)), ((), ())),
                             preferred_element_type=jnp.float32)
    o_ref[0] = xb + scale_ref[0, 0] * comb_t


def kernel(x, scale, Wq, bq, Wk, bk, Wv, bv):
    B_, C_, H_, W_ = x.shape
    N = H_ * W_
    D_ = Wq.shape[0]
    xr = x.reshape(B_, C_, N)
    tc1 = functools.partial(_tc1_body, n=N, d=D_, k=_K)
    v_rows, idx, w = pl.pallas_call(
        tc1,
        grid=(B_,),
        in_specs=[
            pl.BlockSpec((1, C_, N), lambda b: (b, 0, 0)),
            pl.BlockSpec((D_, C_), lambda b: (0, 0)),
            pl.BlockSpec((D_, 1), lambda b: (0, 0)),
            pl.BlockSpec((D_, C_), lambda b: (0, 0)),
            pl.BlockSpec((D_, 1), lambda b: (0, 0)),
            pl.BlockSpec((C_, C_), lambda b: (0, 0)),
            pl.BlockSpec((1, C_), lambda b: (0, 0)),
        ],
        out_specs=[
            pl.BlockSpec((1, N, 128), lambda b: (b, 0, 0)),
            pl.BlockSpec((1, N, _K), lambda b: (b, 0, 0)),
            pl.BlockSpec((1, N, _K), lambda b: (b, 0, 0)),
        ],
        out_shape=[
            jax.ShapeDtypeStruct((B_, N, 128), jnp.float32),
            jax.ShapeDtypeStruct((B_, N, _K), jnp.int32),
            jax.ShapeDtypeStruct((B_, N, _K), jnp.float32),
        ],
    )(xr, Wq, bq.reshape(D_, 1), Wk, bk.reshape(D_, 1), Wv, bv.reshape(1, C_))

    rows_total = B_ * N
    sc = _make_sc_combine(rows_total, C_, _K, 32)
    comb = sc(idx.reshape(rows_total * _K), w.reshape(rows_total * _K),
              v_rows.reshape(rows_total, 128))
    comb = comb.reshape(B_, N, 128)

    tc3 = functools.partial(_tc3_body, n=N, c=C_)
    out = pl.pallas_call(
        tc3,
        grid=(B_,),
        in_specs=[
            pl.BlockSpec((1, C_, N), lambda b: (b, 0, 0)),
            pl.BlockSpec((1, 1), lambda b: (0, 0)),
            pl.BlockSpec((1, N, 128), lambda b: (b, 0, 0)),
        ],
        out_specs=pl.BlockSpec((1, C_, N), lambda b: (b, 0, 0)),
        out_shape=jax.ShapeDtypeStruct((B_, C_, N), jnp.float32),
    )(xr, scale.reshape(1, 1), comb)
    return out.reshape(B_, C_, H_, W_)


# speedup vs baseline: 6.3017x; 6.3017x over previous
import functools

import jax
import jax.numpy as jnp
from jax import lax
from jax.experimental import pallas as pl

_K = 8


def _routing_body(x_ref, scale_ref, wq_ref, bq_ref, wk_ref, bk_ref,
                  wv_ref, bv_ref, o_ref, *, n, d, k, bps):
    for sb in range(bps):
        _routing_one(x_ref, scale_ref, wq_ref, bq_ref, wk_ref, bk_ref,
                     wv_ref, bv_ref, o_ref, sb, n=n, d=d, k=k)


def _routing_one(x_ref, scale_ref, wq_ref, bq_ref, wk_ref, bk_ref,
                 wv_ref, bv_ref, o_ref, sb, *, n, d, k):
    xb = x_ref[sb]
    inv_s = 1.0 / (d ** 0.5)
    qT = (jnp.dot(wq_ref[...], xb, preferred_element_type=jnp.float32)
          + bq_ref[...]) * inv_s
    kT = jnp.dot(wk_ref[...], xb, preferred_element_type=jnp.float32) + bk_ref[...]
    vT = jnp.dot(wv_ref[...], xb, preferred_element_type=jnp.float32) + bv_ref[...]

    sim = lax.dot_general(qT, kT, (((0,), (0,)), ((), ())),
                          preferred_element_type=jnp.float32)
    row = lax.broadcasted_iota(jnp.int32, (n, n), 0)
    col = lax.broadcasted_iota(jnp.int32, (n, n), 1)
    sim = jnp.where(row == col, sim - 1e9, sim)

    work = sim
    m0 = None
    for it in range(k - 1):
        m = jnp.max(work, axis=1, keepdims=True)
        if it == 0:
            m0 = m
        work = jnp.where(work == m, -jnp.inf, work)
    m_last = jnp.max(work, axis=1, keepdims=True)

    e = jnp.where((work == -jnp.inf) | (work == m_last),
                  jnp.exp(sim - m0), 0.0)
    ones_row = jnp.ones((1, n), jnp.float32)
    denom = lax.dot_general(ones_row, e, (((1,), (1,)), ((), ())),
                            preferred_element_type=jnp.float32)
    comb = lax.dot_general(vT, e, (((1,), (1,)), ((), ())),
                           preferred_element_type=jnp.float32)
    o_ref[sb] = xb + (scale_ref[0, 0] / denom) * comb


def kernel(x, scale, Wq, bq, Wk, bk, Wv, bv):
    B_, C_, H_, W_ = x.shape
    N = H_ * W_
    D_ = Wq.shape[0]
    BPS = 2
    xr = x.reshape(B_, C_, N)
    body = functools.partial(_routing_body, n=N, d=D_, k=_K, bps=BPS)
    out = pl.pallas_call(
        body,
        grid=(B_ // BPS,),
        in_specs=[
            pl.BlockSpec((BPS, C_, N), lambda b: (b, 0, 0)),
            pl.BlockSpec((1, 1), lambda b: (0, 0)),
            pl.BlockSpec((D_, C_), lambda b: (0, 0)),
            pl.BlockSpec((D_, 1), lambda b: (0, 0)),
            pl.BlockSpec((D_, C_), lambda b: (0, 0)),
            pl.BlockSpec((D_, 1), lambda b: (0, 0)),
            pl.BlockSpec((C_, C_), lambda b: (0, 0)),
            pl.BlockSpec((C_, 1), lambda b: (0, 0)),
        ],
        out_specs=pl.BlockSpec((BPS, C_, N), lambda b: (b, 0, 0)),
        out_shape=jax.ShapeDtypeStruct((B_, C_, N), jnp.float32),
    )(xr, scale.reshape(1, 1), Wq, bq.reshape(D_, 1), Wk, bk.reshape(D_, 1),
      Wv, bv.reshape(C_, 1))
    return out.reshape(B_, C_, H_, W_)
